# Initial kernel scaffold; baseline (speedup 1.0000x reference)
import jax
import jax.numpy as jnp
from jax.experimental import pallas as pl


def kernel(x, router_w, w_gate, w_up, w_down):
    raise NotImplementedError("write your pallas kernel here")


# speedup vs baseline: 1.2691x; 1.2691x over previous
import functools

import jax
import jax.numpy as jnp
from jax import lax
from jax.experimental import pallas as pl
from jax.experimental.pallas import tpu as pltpu
from jax.experimental.pallas import tpu_sc as plsc

T = 256
D = 2048
E = 64
K = 8
FF = 1024

BR = 64
MAXBLK = T // BR
FFB = 512
FB = FF // FFB

_NC = 2
_NS = 16
_NW = _NC * _NS
_EW = E // _NW


def _router_body(x_ref, rw_ref, wt_ref, pos_ref, cnt_ref):
    xx = x_ref[...]
    rw = rw_ref[...]
    lg = lax.dot_general(rw, xx, (((1,), (1,)), ((), ())),
                         preferred_element_type=jnp.float32)
    mx = jnp.max(lg, axis=0, keepdims=True)
    pe = jnp.exp(lg - mx)
    pr = pe / jnp.sum(pe, axis=0, keepdims=True)

    ei = lax.broadcasted_iota(jnp.int32, (E, E), 0)
    ej = lax.broadcasted_iota(jnp.int32, (E, E), 1)
    lower = (ej <= ei).astype(jnp.float32)

    pm = pr
    wt = jnp.zeros((E, T), jnp.float32)
    msk = jnp.zeros((E, T), jnp.float32)
    for _ in range(K):
        cm = jnp.max(pm, axis=0, keepdims=True)
        sel = (pm == cm).astype(jnp.float32)
        csum = lax.dot_general(lower, sel, (((1,), (0,)), ((), ())),
                               preferred_element_type=jnp.float32)
        first = sel * (csum == 1.0).astype(jnp.float32)
        wt = wt + pr * first
        msk = msk + first
        pm = pm - first * 2.0

    ti = lax.broadcasted_iota(jnp.int32, (T, T), 0)
    tj = lax.broadcasted_iota(jnp.int32, (T, T), 1)
    upper = (ti < tj).astype(jnp.float32)
    posf = lax.dot_general(msk, upper, (((1,), (0,)), ((), ())),
                           preferred_element_type=jnp.float32)

    wt_ref[...] = wt
    pos_ref[...] = posf.astype(jnp.int32)
    cnt_ref[...] = jnp.sum(msk, axis=1, keepdims=True).astype(jnp.int32)


def _router(x, router_w):
    return pl.pallas_call(
        _router_body,
        out_shape=(
            jax.ShapeDtypeStruct((E, T), jnp.float32),
            jax.ShapeDtypeStruct((E, T), jnp.int32),
            jax.ShapeDtypeStruct((E, 1), jnp.int32),
        ),
    )(x, router_w)


def _dispatch_body(wt_hbm, pos_hbm, tl_hbm, wl_hbm, w_v, p_v, tl_v, wl_v):
    c = lax.axis_index("c")
    s = lax.axis_index("s")
    wid = s * _NC + c
    base = wid * (_EW * T)

    pltpu.sync_copy(wt_hbm.at[pl.ds(base, _EW * T)], w_v)
    pltpu.sync_copy(pos_hbm.at[pl.ds(base, _EW * T)], p_v)

    zi = jnp.zeros((16,), jnp.int32)
    zf = jnp.zeros((16,), jnp.float32)
    for i in range(_EW * T // 16):
        tl_v[pl.ds(i * 16, 16)] = zi
        wl_v[pl.ds(i * 16, 16)] = zf
    for le in range(_EW):
        for i in range(T // 16):
            w = w_v[pl.ds(le * T + i * 16, 16)]
            p = p_v[pl.ds(le * T + i * 16, 16)]
            t = lax.iota(jnp.int32, 16) + (i * 16)
            m = w > 0.0
            dest = p + (le * T)
            plsc.store_scatter(tl_v, [dest], t, mask=m)
            plsc.store_scatter(wl_v, [dest], w, mask=m)

    pltpu.sync_copy(tl_v, tl_hbm.at[pl.ds(base, _EW * T)])
    pltpu.sync_copy(wl_v, wl_hbm.at[pl.ds(base, _EW * T)])


def _dispatch(wt, pos):
    mesh = plsc.VectorSubcoreMesh(core_axis_name="c", subcore_axis_name="s")
    return pl.kernel(
        _dispatch_body,
        mesh=mesh,
        out_type=[
            jax.ShapeDtypeStruct((E * T,), jnp.int32),
            jax.ShapeDtypeStruct((E * T,), jnp.float32),
        ],
        scratch_types=[
            pltpu.VMEM((_EW * T,), jnp.float32),
            pltpu.VMEM((_EW * T,), jnp.int32),
            pltpu.VMEM((_EW * T,), jnp.int32),
            pltpu.VMEM((_EW * T,), jnp.float32),
        ],
        compiler_params=pltpu.CompilerParams(needs_layout_passes=False),
    )(wt.reshape(E * T), pos.reshape(E * T))


def _ffn_body(cnt_ref, tl_ref, wl_ref, x_ref, wg_ref, wu_ref, wd_ref,
              out_ref, xg_s, y_s):
    e = pl.program_id(0)
    fb = pl.program_id(1)
    n = cnt_ref[e]

    @pl.when((e == 0) & (fb == 0))
    def _init():
        out_ref[...] = jnp.zeros_like(out_ref)

    wg = wg_ref[0]
    wu = wu_ref[0]
    wd = wd_ref[0]

    for j in range(MAXBLK):
        @pl.when(j * BR < n)
        def _block(j=j):
            js = slice(j * BR, (j + 1) * BR)
            tl_row = tl_ref[0, :, js]
            st = (lax.broadcasted_iota(jnp.int32, (T, BR), 0)
                  == tl_row).astype(jnp.float32)

            @pl.when(fb == 0)
            def _gather():
                xg_s[js, :] = lax.dot_general(
                    st, x_ref[...], (((0,), (0,)), ((), ())),
                    preferred_element_type=jnp.float32)

            xg = xg_s[js, :]
            g = lax.dot_general(xg, wg, (((1,), (1,)), ((), ())),
                                preferred_element_type=jnp.float32)
            u = lax.dot_general(xg, wu, (((1,), (1,)), ((), ())),
                                preferred_element_type=jnp.float32)
            h = g * jax.nn.sigmoid(g) * u
            yj = lax.dot_general(h, wd, (((1,), (1,)), ((), ())),
                                 preferred_element_type=jnp.float32)

            @pl.when(fb == 0)
            def _seed_y():
                y_s[js, :] = yj

            @pl.when(fb != 0)
            def _acc_y():
                y_s[js, :] += yj

            @pl.when(fb == FB - 1)
            def _combine():
                sw = st * wl_ref[0, :, js]
                out_ref[...] += lax.dot_general(
                    sw, y_s[js, :], (((1,), (0,)), ((), ())),
                    preferred_element_type=jnp.float32)


def _ffn(cnt, tl3, wl3, x, w_gate, w_up, w_down):
    grid_spec = pltpu.PrefetchScalarGridSpec(
        num_scalar_prefetch=1,
        grid=(E, FB),
        in_specs=[
            pl.BlockSpec((1, 1, T), lambda e, fb, c: (e, 0, 0)),
            pl.BlockSpec((1, 1, T), lambda e, fb, c: (e, 0, 0)),
            pl.BlockSpec((T, D), lambda e, fb, c: (0, 0)),
            pl.BlockSpec((1, FFB, D), lambda e, fb, c: (e, fb, 0)),
            pl.BlockSpec((1, FFB, D), lambda e, fb, c: (e, fb, 0)),
            pl.BlockSpec((1, D, FFB), lambda e, fb, c: (e, 0, fb)),
        ],
        out_specs=pl.BlockSpec((T, D), lambda e, fb, c: (0, 0)),
        scratch_shapes=[
            pltpu.VMEM((T, D), jnp.float32),
            pltpu.VMEM((T, D), jnp.float32),
        ],
    )
    return pl.pallas_call(
        _ffn_body,
        grid_spec=grid_spec,
        out_shape=jax.ShapeDtypeStruct((T, D), jnp.float32),
        compiler_params=pltpu.CompilerParams(
            dimension_semantics=("arbitrary", "arbitrary")),
    )(cnt, tl3, wl3, x, w_gate, w_up, w_down)


def kernel(x, router_w, w_gate, w_up, w_down):
    wt, pos, cnt = _router(x, router_w)
    tl, wl = _dispatch(wt, pos)
    tl = tl.reshape(E, T)
    wl = wl.reshape(E, T)
    out = _ffn(cnt.reshape(E), tl.reshape(E, 1, T), wl.reshape(E, 1, T),
               x, w_gate, w_up, w_down)
    return out


# speedup vs baseline: 1.3044x; 1.0278x over previous
import functools

import jax
import jax.numpy as jnp
from jax import lax
from jax.experimental import pallas as pl
from jax.experimental.pallas import tpu as pltpu
from jax.experimental.pallas import tpu_sc as plsc

T = 256
D = 2048
E = 64
K = 8
FF = 1024

BR = 64
MAXBLK = T // BR
FFB = 512
FB = FF // FFB

_NC = 2
_NS = 16
_NW = _NC * _NS
_EW = E // _NW


def _router_body(x_ref, rw_ref, wt_ref, pos_ref, cnt_ref):
    xx = x_ref[...]
    rw = rw_ref[...]
    lg = lax.dot_general(rw, xx, (((1,), (1,)), ((), ())),
                         preferred_element_type=jnp.float32)
    mx = jnp.max(lg, axis=0, keepdims=True)
    pe = jnp.exp(lg - mx)
    pr = pe / jnp.sum(pe, axis=0, keepdims=True)

    ei = lax.broadcasted_iota(jnp.int32, (E, E), 0)
    ej = lax.broadcasted_iota(jnp.int32, (E, E), 1)
    lower = (ej <= ei).astype(jnp.float32)

    pm = pr
    wt = jnp.zeros((E, T), jnp.float32)
    msk = jnp.zeros((E, T), jnp.float32)
    for _ in range(K):
        cm = jnp.max(pm, axis=0, keepdims=True)
        sel = (pm == cm).astype(jnp.float32)
        csum = lax.dot_general(lower, sel, (((1,), (0,)), ((), ())),
                               preferred_element_type=jnp.float32)
        first = sel * (csum == 1.0).astype(jnp.float32)
        wt = wt + pr * first
        msk = msk + first
        pm = pm - first * 2.0

    ti = lax.broadcasted_iota(jnp.int32, (T, T), 0)
    tj = lax.broadcasted_iota(jnp.int32, (T, T), 1)
    upper = (ti < tj).astype(jnp.float32)
    posf = lax.dot_general(msk, upper, (((1,), (0,)), ((), ())),
                           preferred_element_type=jnp.float32)

    wt_ref[...] = wt
    pos_ref[...] = posf.astype(jnp.int32)
    cnt_ref[...] = jnp.sum(msk, axis=1, keepdims=True).astype(jnp.int32)


def _router(x, router_w):
    return pl.pallas_call(
        _router_body,
        out_shape=(
            jax.ShapeDtypeStruct((E, T), jnp.float32),
            jax.ShapeDtypeStruct((E, T), jnp.int32),
            jax.ShapeDtypeStruct((E, 1), jnp.int32),
        ),
    )(x, router_w)


def _dispatch_body(wt_hbm, pos_hbm, tl_hbm, wl_hbm, w_v, p_v, tl_v, wl_v):
    c = lax.axis_index("c")
    s = lax.axis_index("s")
    wid = s * _NC + c
    base = wid * (_EW * T)

    pltpu.sync_copy(wt_hbm.at[pl.ds(base, _EW * T)], w_v)
    pltpu.sync_copy(pos_hbm.at[pl.ds(base, _EW * T)], p_v)

    zi = jnp.zeros((16,), jnp.int32)
    zf = jnp.zeros((16,), jnp.float32)
    for i in range(_EW * T // 16):
        tl_v[pl.ds(i * 16, 16)] = zi
        wl_v[pl.ds(i * 16, 16)] = zf
    for le in range(_EW):
        for i in range(T // 16):
            w = w_v[pl.ds(le * T + i * 16, 16)]
            p = p_v[pl.ds(le * T + i * 16, 16)]
            t = lax.iota(jnp.int32, 16) + (i * 16)
            m = w > 0.0
            dest = p + (le * T)
            plsc.store_scatter(tl_v, [dest], t, mask=m)
            plsc.store_scatter(wl_v, [dest], w, mask=m)

    pltpu.sync_copy(tl_v, tl_hbm.at[pl.ds(base, _EW * T)])
    pltpu.sync_copy(wl_v, wl_hbm.at[pl.ds(base, _EW * T)])


def _dispatch(wt, pos):
    mesh = plsc.VectorSubcoreMesh(core_axis_name="c", subcore_axis_name="s")
    return pl.kernel(
        _dispatch_body,
        mesh=mesh,
        out_type=[
            jax.ShapeDtypeStruct((E * T,), jnp.int32),
            jax.ShapeDtypeStruct((E * T,), jnp.float32),
        ],
        scratch_types=[
            pltpu.VMEM((_EW * T,), jnp.float32),
            pltpu.VMEM((_EW * T,), jnp.int32),
            pltpu.VMEM((_EW * T,), jnp.int32),
            pltpu.VMEM((_EW * T,), jnp.float32),
        ],
        compiler_params=pltpu.CompilerParams(needs_layout_passes=False),
    )(wt.reshape(E * T), pos.reshape(E * T))


def _ffn_body(cnt_ref, tl_ref, wl_ref, x_ref, wg_ref, wu_ref, wd_ref,
              out_ref, xg_s, y_s):
    e = pl.program_id(0)
    fb = pl.program_id(1)
    n = cnt_ref[e]

    @pl.when((e == 0) & (fb == 0))
    def _init():
        out_ref[...] = jnp.zeros_like(out_ref)

    wg = wg_ref[0].astype(jnp.bfloat16)
    wu = wu_ref[0].astype(jnp.bfloat16)
    wd = wd_ref[0].astype(jnp.bfloat16)

    for j in range(MAXBLK):
        @pl.when(j * BR < n)
        def _block(j=j):
            js = slice(j * BR, (j + 1) * BR)
            tl_row = tl_ref[0, :, js]
            st = (lax.broadcasted_iota(jnp.int32, (T, BR), 0)
                  == tl_row).astype(jnp.float32)

            @pl.when(fb == 0)
            def _gather():
                xg_s[js, :] = lax.dot_general(
                    st, x_ref[...], (((0,), (0,)), ((), ())),
                    preferred_element_type=jnp.float32)

            xg = xg_s[js, :].astype(jnp.bfloat16)
            g = lax.dot_general(xg, wg, (((1,), (1,)), ((), ())),
                                preferred_element_type=jnp.float32)
            u = lax.dot_general(xg, wu, (((1,), (1,)), ((), ())),
                                preferred_element_type=jnp.float32)
            h = (g * jax.nn.sigmoid(g) * u).astype(jnp.bfloat16)
            yj = lax.dot_general(h, wd, (((1,), (1,)), ((), ())),
                                 preferred_element_type=jnp.float32)

            @pl.when(fb == 0)
            def _seed_y():
                y_s[js, :] = yj

            @pl.when(fb != 0)
            def _acc_y():
                y_s[js, :] += yj

            @pl.when(fb == FB - 1)
            def _combine():
                sw = st * wl_ref[0, :, js]
                out_ref[...] += lax.dot_general(
                    sw, y_s[js, :], (((1,), (0,)), ((), ())),
                    preferred_element_type=jnp.float32)


def _ffn(cnt, tl3, wl3, x, w_gate, w_up, w_down):
    grid_spec = pltpu.PrefetchScalarGridSpec(
        num_scalar_prefetch=1,
        grid=(E, FB),
        in_specs=[
            pl.BlockSpec((1, 1, T), lambda e, fb, c: (e, 0, 0)),
            pl.BlockSpec((1, 1, T), lambda e, fb, c: (e, 0, 0)),
            pl.BlockSpec((T, D), lambda e, fb, c: (0, 0)),
            pl.BlockSpec((1, FFB, D), lambda e, fb, c: (e, fb, 0)),
            pl.BlockSpec((1, FFB, D), lambda e, fb, c: (e, fb, 0)),
            pl.BlockSpec((1, D, FFB), lambda e, fb, c: (e, 0, fb)),
        ],
        out_specs=pl.BlockSpec((T, D), lambda e, fb, c: (0, 0)),
        scratch_shapes=[
            pltpu.VMEM((T, D), jnp.float32),
            pltpu.VMEM((T, D), jnp.float32),
        ],
    )
    return pl.pallas_call(
        _ffn_body,
        grid_spec=grid_spec,
        out_shape=jax.ShapeDtypeStruct((T, D), jnp.float32),
        compiler_params=pltpu.CompilerParams(
            dimension_semantics=("arbitrary", "arbitrary")),
    )(cnt, tl3, wl3, x, w_gate, w_up, w_down)


def kernel(x, router_w, w_gate, w_up, w_down):
    wt, pos, cnt = _router(x, router_w)
    tl, wl = _dispatch(wt, pos)
    tl = tl.reshape(E, T)
    wl = wl.reshape(E, T)
    out = _ffn(cnt.reshape(E), tl.reshape(E, 1, T), wl.reshape(E, 1, T),
               x, w_gate, w_up, w_down)
    return out
